# baseline (speedup 1.0000x reference)
import dataclasses
import functools

import jax
import jax.numpy as jnp
from jax import lax
from jax.experimental import pallas as pl
from jax.experimental.pallas import tpu as pltpu
from jax.experimental.pallas import tpu_sc as plsc

N = 10000
E = 320000
D = 128
DH = D // 2
NC = 2
NS = 16
NW = NC * NS
CHUNK = 128
NCHUNK = 80
EPT = NCHUNK * CHUNK
EP = NW * EPT
GARBAGE = N
NP = 10112
RPT = NP // NS
ZROWS = 8
L = 16

_MESH = plsc.VectorSubcoreMesh(core_axis_name="c", subcore_axis_name="s")

_SC_PARAMS = pltpu.CompilerParams()
if "needs_layout_passes" in pltpu.CompilerParams.__dataclass_fields__:
    _SC_PARAMS = dataclasses.replace(_SC_PARAMS, needs_layout_passes=False)
_SC_AGG_PARAMS = pltpu.CompilerParams()
if "use_tc_tiling_on_sc" in pltpu.CompilerParams.__dataclass_fields__:
    _SC_AGG_PARAMS = dataclasses.replace(_SC_AGG_PARAMS,
                                         use_tc_tiling_on_sc=False)


@functools.partial(
    pl.kernel,
    out_type=jax.ShapeDtypeStruct((NW, NP), jnp.float32),
    mesh=_MESH,
    scratch_types=[
        pltpu.VMEM((EPT,), jnp.int32),
        pltpu.VMEM((NP,), jnp.float32),
    ],
    compiler_params=_SC_PARAMS,
)
def _sc_degree(dst_hbm, out_hbm, dst_v, cnt_v):
    c = lax.axis_index("c")
    s = lax.axis_index("s")
    wid = c * NS + s

    @pl.loop(0, NP, step=L)
    def _(r):
        cnt_v[pl.ds(r, L)] = jnp.zeros((L,), jnp.float32)

    pltpu.sync_copy(dst_hbm.at[wid], dst_v)

    ones = jnp.full((L,), 1.0, jnp.float32)

    @pl.loop(0, EPT, step=L)
    def _(g):
        idx = dst_v[pl.ds(g, L)]
        plsc.addupdate_scatter(cnt_v, [idx], ones)

    pltpu.sync_copy(cnt_v, out_hbm.at[wid])


@functools.partial(
    pl.kernel,
    out_type=[jax.ShapeDtypeStruct((NC * NP, DH), jnp.float32),
              jax.ShapeDtypeStruct((NC * NP, DH), jnp.float32)],
    mesh=_MESH,
    scratch_types=[
        pltpu.VMEM((NCHUNK, CHUNK), jnp.int32),
        pltpu.VMEM((NCHUNK, CHUNK), jnp.int32),
        pltpu.VMEM((CHUNK, DH), jnp.float32),
        pltpu.VMEM((CHUNK, DH), jnp.float32),
        pltpu.VMEM((ZROWS, DH), jnp.float32),
        pltpu.VMEM_SHARED((NP, DH), jnp.float32),
        pltpu.SemaphoreType.DMA,
        pltpu.SemaphoreType.DMA,
    ],
    compiler_params=_SC_AGG_PARAMS,
)
def _sc_aggregate(ylo_hbm, yhi_hbm, src_hbm, dst_hbm, outlo_hbm, outhi_hbm,
                  src_v, dst_v, rows0_v, rows1_v, zbuf_v, acc_sh,
                  gsem0, gsem1):
    c = lax.axis_index("c")
    s = lax.axis_index("s")
    wid = c * NS + s
    r0 = s * RPT

    @pl.loop(0, ZROWS)
    def _(r):
        @pl.loop(0, DH, step=L)
        def _(c0):
            zbuf_v[r, pl.ds(c0, L)] = jnp.zeros((L,), jnp.float32)

    pltpu.sync_copy(src_hbm.at[wid], src_v)
    pltpu.sync_copy(dst_hbm.at[wid], dst_v)

    for y_hbm, out_hbm in ((ylo_hbm, outlo_hbm), (yhi_hbm, outhi_hbm)):
        @pl.loop(0, RPT, step=ZROWS)
        def _(i):
            pltpu.sync_copy(zbuf_v, acc_sh.at[pl.ds(r0 + i, ZROWS)])

        plsc.subcore_barrier()

        pltpu.async_copy(y_hbm.at[src_v.at[0]], rows0_v, gsem0)

        @pl.loop(0, NCHUNK - 2, step=2)
        def _(j):
            pltpu.async_copy(y_hbm.at[src_v.at[j + 1]], rows1_v, gsem1)
            pltpu.make_async_copy(y_hbm.at[src_v.at[j]], rows0_v, gsem0).wait()
            pltpu.sync_copy(rows0_v, acc_sh.at[dst_v.at[j]], add=True)
            pltpu.async_copy(y_hbm.at[src_v.at[j + 2]], rows0_v, gsem0)
            pltpu.make_async_copy(y_hbm.at[src_v.at[j + 1]], rows1_v,
                                  gsem1).wait()
            pltpu.sync_copy(rows1_v, acc_sh.at[dst_v.at[j + 1]], add=True)

        pltpu.async_copy(y_hbm.at[src_v.at[NCHUNK - 1]], rows1_v, gsem1)
        pltpu.make_async_copy(y_hbm.at[src_v.at[NCHUNK - 2]], rows0_v,
                              gsem0).wait()
        pltpu.sync_copy(rows0_v, acc_sh.at[dst_v.at[NCHUNK - 2]], add=True)
        pltpu.make_async_copy(y_hbm.at[src_v.at[NCHUNK - 1]], rows1_v,
                              gsem1).wait()
        pltpu.sync_copy(rows1_v, acc_sh.at[dst_v.at[NCHUNK - 1]], add=True)

        plsc.subcore_barrier()
        pltpu.sync_copy(acc_sh.at[pl.ds(r0, RPT)],
                        out_hbm.at[pl.ds(c * NP + r0, RPT)])


BM = 1024

_DOT = dict(precision=lax.Precision.HIGHEST, preferred_element_type=jnp.float32)


def _mm_body(x_ref, w_ref, o_ref):
    o_ref[...] = lax.dot_general(x_ref[...], w_ref[...],
                                 (((1,), (1,)), ((), ())), **_DOT)


def _matmul(x, w):
    return pl.pallas_call(
        _mm_body,
        grid=(NP // BM + 1,),
        in_specs=[pl.BlockSpec((BM, D), lambda i: (i, 0)),
                  pl.BlockSpec((D, D), lambda i: (0, 0))],
        out_specs=pl.BlockSpec((BM, D), lambda i: (i, 0)),
        out_shape=jax.ShapeDtypeStruct((N, D), jnp.float32),
    )(x, w)


def _dis_y_body(degp_ref, t_ref, dis_ref, ylo_ref, yhi_ref):
    deg = jnp.sum(degp_ref[...], axis=0) + 1.0
    dis = lax.rsqrt(deg)[:, None]
    dis_ref[...] = dis
    y = t_ref[...] * dis
    ylo_ref[...] = y[:, :DH]
    yhi_ref[...] = y[:, DH:]


def _dis_y(degp, t):
    return pl.pallas_call(
        _dis_y_body,
        grid=(NP // BM + 1,),
        in_specs=[pl.BlockSpec((NW, BM), lambda i: (0, i)),
                  pl.BlockSpec((BM, D), lambda i: (i, 0))],
        out_specs=[pl.BlockSpec((BM, 1), lambda i: (i, 0)),
                   pl.BlockSpec((BM, DH), lambda i: (i, 0)),
                   pl.BlockSpec((BM, DH), lambda i: (i, 0))],
        out_shape=[jax.ShapeDtypeStruct((NP, 1), jnp.float32),
                   jax.ShapeDtypeStruct((N, DH), jnp.float32),
                   jax.ShapeDtypeStruct((N, DH), jnp.float32)],
    )(degp, t)


def _agg_sum(splo_ref, sphi_ref, ylo_ref, yhi_ref):
    lo = splo_ref[0] + splo_ref[1] + ylo_ref[...]
    hi = sphi_ref[0] + sphi_ref[1] + yhi_ref[...]
    return jnp.concatenate([lo, hi], axis=1)


def _mid_body(splo_ref, sphi_ref, ylo_ref, yhi_ref, dis_ref, b_ref, w_ref,
              olo_ref, ohi_ref):
    dis = dis_ref[...]
    agg = _agg_sum(splo_ref, sphi_ref, ylo_ref, yhi_ref)
    h = jnp.maximum(agg * dis + b_ref[...][None, :], 0.0)
    t = lax.dot_general(h, w_ref[...], (((1,), (1,)), ((), ())), **_DOT)
    y = t * dis
    olo_ref[...] = y[:, :DH]
    ohi_ref[...] = y[:, DH:]


def _mid_layer(splo, sphi, ylo, yhi, dis, b, w):
    return pl.pallas_call(
        _mid_body,
        grid=(NP // BM + 1,),
        in_specs=[pl.BlockSpec((NC, BM, DH), lambda i: (0, i, 0)),
                  pl.BlockSpec((NC, BM, DH), lambda i: (0, i, 0)),
                  pl.BlockSpec((BM, DH), lambda i: (i, 0)),
                  pl.BlockSpec((BM, DH), lambda i: (i, 0)),
                  pl.BlockSpec((BM, 1), lambda i: (i, 0)),
                  pl.BlockSpec((D,), lambda i: (0,)),
                  pl.BlockSpec((D, D), lambda i: (0, 0))],
        out_specs=[pl.BlockSpec((BM, DH), lambda i: (i, 0)),
                   pl.BlockSpec((BM, DH), lambda i: (i, 0))],
        out_shape=[jax.ShapeDtypeStruct((N, DH), jnp.float32),
                   jax.ShapeDtypeStruct((N, DH), jnp.float32)],
    )(splo, sphi, ylo, yhi, dis, b, w)


def _final_body(splo_ref, sphi_ref, ylo_ref, yhi_ref, dis_ref, b_ref, o_ref):
    agg = _agg_sum(splo_ref, sphi_ref, ylo_ref, yhi_ref)
    o_ref[...] = jnp.maximum(agg * dis_ref[...] + b_ref[...][None, :], 0.0)


def _final_layer(splo, sphi, ylo, yhi, dis, b):
    return pl.pallas_call(
        _final_body,
        grid=(NP // BM + 1,),
        in_specs=[pl.BlockSpec((NC, BM, DH), lambda i: (0, i, 0)),
                  pl.BlockSpec((NC, BM, DH), lambda i: (0, i, 0)),
                  pl.BlockSpec((BM, DH), lambda i: (i, 0)),
                  pl.BlockSpec((BM, DH), lambda i: (i, 0)),
                  pl.BlockSpec((BM, 1), lambda i: (i, 0)),
                  pl.BlockSpec((D,), lambda i: (0,))],
        out_specs=pl.BlockSpec((BM, D), lambda i: (i, 0)),
        out_shape=jax.ShapeDtypeStruct((N, D), jnp.float32),
    )(splo, sphi, ylo, yhi, dis, b)


@jax.jit
def kernel(x, edge_index, W1, b1, W2, b2):
    ei = edge_index.astype(jnp.int32)
    pad = EP - E
    src = jnp.concatenate([ei[0], jnp.zeros((pad,), jnp.int32)])
    dst = jnp.concatenate([ei[1], jnp.full((pad,), GARBAGE, jnp.int32)])
    src = src.reshape(NW, NCHUNK, CHUNK)
    dst = dst.reshape(NW, NCHUNK, CHUNK)

    degp = _sc_degree(dst.reshape(NW, EPT))
    t1 = _matmul(x, W1)
    dis, y1lo, y1hi = _dis_y(degp, t1)

    s1lo, s1hi = _sc_aggregate(y1lo, y1hi, src, dst)
    s1lo = s1lo.reshape(NC, NP, DH)
    s1hi = s1hi.reshape(NC, NP, DH)
    y2lo, y2hi = _mid_layer(s1lo, s1hi, y1lo, y1hi, dis, b1, W2)

    s2lo, s2hi = _sc_aggregate(y2lo, y2hi, src, dst)
    s2lo = s2lo.reshape(NC, NP, DH)
    s2hi = s2hi.reshape(NC, NP, DH)
    return _final_layer(s2lo, s2hi, y2lo, y2hi, dis, b2)
